# Initial kernel scaffold; baseline (speedup 1.0000x reference)
import jax
import jax.numpy as jnp
from jax.experimental import pallas as pl


def kernel(x, edge_index, W1l, b1l, W1r, W2l, b2l, W2r, Wlin, blin):
    raise NotImplementedError("write your pallas kernel here")


# speedup vs baseline: 2.9358x; 2.9358x over previous
import functools

import jax
import jax.numpy as jnp
from jax import lax
from jax.experimental import pallas as pl
from jax.experimental.pallas import tpu as pltpu
from jax.experimental.pallas import tpu_sc as plsc

N = 10000
D = 128
NC = 2
NS = 16
NW = NC * NS
C = 128
N_PAD = 10240
ROWS_PER_TILE = N_PAD // NS


def _sc_agg_body(n_chunks, with_deg, *refs):
    if with_deg:
        (x_hbm, src_hbm, dst_hbm, agg_out, deg_out,
         agg_sh, deg_sh, sidx, didx, rows, zbuf, ones, sem) = refs
    else:
        (x_hbm, src_hbm, dst_hbm, agg_out,
         agg_sh, sidx, didx, rows, zbuf, sem) = refs
    cid = lax.axis_index("c")
    sid = lax.axis_index("s")
    wid = cid * NS + sid

    zeros16 = jnp.zeros((16,), jnp.float32)

    def zrow(i, carry):
        for j in range(D // 16):
            zbuf[i, pl.ds(j * 16, 16)] = zeros16
        return carry

    lax.fori_loop(0, C, zrow, 0)
    if with_deg:
        ones16 = jnp.full((16,), 1.0, jnp.float32)
        for j in range(C // 16):
            ones[pl.ds(j * 16, 16)] = ones16

    r0 = sid * ROWS_PER_TILE
    for k in range(ROWS_PER_TILE // C):
        pltpu.sync_copy(zbuf, agg_sh.at[pl.ds(r0 + k * C, C)])
    if with_deg:
        for k in range(ROWS_PER_TILE // C):
            pltpu.sync_copy(zbuf.at[0, pl.ds(0, C)],
                            deg_sh.at[pl.ds(r0 + k * C, C)])
    plsc.subcore_barrier()

    epw = n_chunks * C

    def chunk(i, carry):
        base = wid * epw + i * C
        pltpu.sync_copy(src_hbm.at[pl.ds(base, C)], sidx)
        pltpu.sync_copy(dst_hbm.at[pl.ds(base, C)], didx)
        pltpu.async_copy(x_hbm.at[sidx], rows, sem).wait()
        pltpu.sync_copy(rows, agg_sh.at[didx], add=True)
        if with_deg:
            pltpu.sync_copy(ones, deg_sh.at[didx], add=True)
        return carry

    lax.fori_loop(0, n_chunks, chunk, 0)
    plsc.subcore_barrier()

    pltpu.sync_copy(agg_sh.at[pl.ds(r0, ROWS_PER_TILE)],
                    agg_out.at[cid, pl.ds(r0, ROWS_PER_TILE)])
    if with_deg:
        pltpu.sync_copy(deg_sh.at[pl.ds(r0, ROWS_PER_TILE)],
                        deg_out.at[cid, pl.ds(r0, ROWS_PER_TILE)])


def _make_sc_agg(n_chunks, with_deg):
    mesh = plsc.VectorSubcoreMesh(core_axis_name="c", subcore_axis_name="s",
                                  num_cores=NC, num_subcores=NS)
    out_type = [jax.ShapeDtypeStruct((NC, N_PAD, D), jnp.float32)]
    scratch = [
        pltpu.VMEM_SHARED((N_PAD, D), jnp.float32),
    ]
    if with_deg:
        out_type.append(jax.ShapeDtypeStruct((NC, N_PAD), jnp.float32))
        scratch.append(pltpu.VMEM_SHARED((N_PAD,), jnp.float32))
    scratch += [
        pltpu.VMEM((C,), jnp.int32),
        pltpu.VMEM((C,), jnp.int32),
        pltpu.VMEM((C, D), jnp.float32),
        pltpu.VMEM((C, D), jnp.float32),
    ]
    if with_deg:
        scratch.append(pltpu.VMEM((C,), jnp.float32))
    scratch.append(pltpu.SemaphoreType.DMA)

    body = functools.partial(_sc_agg_body, n_chunks, with_deg)
    return pl.kernel(body, out_type=out_type, mesh=mesh,
                     scratch_types=scratch,
                     name=f"sc_agg_deg{int(with_deg)}")


def _tc_layer1(aggp, degp, x, wl, bl, wr, h):
    deg = jnp.maximum(degp[0] + degp[1], 1.0)
    mean = (aggp[0] + aggp[1]) / deg
    acc = jnp.dot(mean, wl[...], preferred_element_type=jnp.float32)
    acc = acc + jnp.dot(x[...], wr[...], preferred_element_type=jnp.float32)
    h[...] = jnp.maximum(acc + bl[...], 0.0)


def _tc_layer2(aggp, degp, h1, wl, bl, wr, wlin, blin, out, emb):
    deg = jnp.maximum(degp[0] + degp[1], 1.0)
    mean = (aggp[0] + aggp[1]) / deg
    acc = jnp.dot(mean, wl[...], preferred_element_type=jnp.float32)
    acc = acc + jnp.dot(h1[...], wr[...], preferred_element_type=jnp.float32)
    e = jnp.maximum(acc + bl[...], 0.0)
    emb[...] = e
    out[...] = jnp.dot(e, wlin[...], preferred_element_type=jnp.float32) + blin[...]


def kernel(x, edge_index, W1l, b1l, W1r, W2l, b2l, W2r, Wlin, blin):
    E = edge_index.shape[1]
    n_chunks = -(-E // (NW * C))
    if n_chunks % 2:
        n_chunks += 1
    e_pad = n_chunks * NW * C
    src = jnp.concatenate(
        [edge_index[0], jnp.zeros((e_pad - E,), jnp.int32)])
    dst = jnp.concatenate(
        [edge_index[1], jnp.full((e_pad - E,), N, jnp.int32)])

    sc1 = _make_sc_agg(n_chunks, True)
    agg1, deg = sc1(x, src, dst)
    deg3 = deg.reshape(NC, N_PAD, 1)

    R = 1000
    grid = (N // R,)
    w_spec = pl.BlockSpec((D, D), lambda i: (0, 0))
    b_spec = pl.BlockSpec((1, D), lambda i: (0, 0))
    agg_spec = pl.BlockSpec((NC, R, D), lambda i: (0, i, 0))
    deg_spec = pl.BlockSpec((NC, R, 1), lambda i: (0, i, 0))
    row_spec = pl.BlockSpec((R, D), lambda i: (i, 0))

    h1 = pl.pallas_call(
        _tc_layer1,
        grid=grid,
        in_specs=[agg_spec, deg_spec, row_spec, w_spec, b_spec, w_spec],
        out_specs=row_spec,
        out_shape=jax.ShapeDtypeStruct((N, D), jnp.float32),
    )(agg1, deg3, x, W1l, b1l.reshape(1, D), W1r)

    sc2 = _make_sc_agg(n_chunks, False)
    (agg2,) = sc2(h1, src, dst)

    out, emb = pl.pallas_call(
        _tc_layer2,
        grid=grid,
        in_specs=[agg_spec, deg_spec, row_spec, w_spec, b_spec, w_spec,
                  w_spec, b_spec],
        out_specs=[row_spec, row_spec],
        out_shape=[jax.ShapeDtypeStruct((N, D), jnp.float32),
                   jax.ShapeDtypeStruct((N, D), jnp.float32)],
    )(agg2, deg3, h1, W2l, b2l.reshape(1, D), W2r, Wlin, blin.reshape(1, D))
    return (out, emb)
